# baseline (speedup 1.0000x reference)
import functools

import jax
import jax.numpy as jnp
from jax import lax
from jax.experimental import pallas as pl
from jax.experimental.pallas import tpu as pltpu
from jax.experimental.pallas import tpu_sc as plsc

NUM_TASKS = 100000
NUM_DBS = 1000
TASK_DIM = 64
DB_DIM = 32
BATCH = 16384

_INFO = plsc.get_sparse_core_info()
NC = _INFO.num_cores
NS = _INFO.num_subcores
NW = NC * NS
QTR = BATCH // 4
LANES = 16
UNROLL = 16


def _sc_gather_t(task_id, db_id, task_t, db_t):
    mesh = plsc.VectorSubcoreMesh(core_axis_name="c", subcore_axis_name="s")

    @functools.partial(
        pl.kernel,
        mesh=mesh,
        out_type=(
            jax.ShapeDtypeStruct((TASK_DIM, BATCH), jnp.float32),
            jax.ShapeDtypeStruct((DB_DIM, BATCH), jnp.float32),
        ),
        scratch_types=[
            pltpu.VMEM((NUM_TASKS,), jnp.float32),
            pltpu.VMEM((NUM_DBS,), jnp.float32),
            pltpu.VMEM((BATCH,), jnp.int32),
            pltpu.VMEM((QTR,), jnp.float32),
            pltpu.VMEM((QTR,), jnp.float32),
            pltpu.SemaphoreType.DMA,
            pltpu.SemaphoreType.DMA,
            pltpu.SemaphoreType.DMA,
            pltpu.SemaphoreType.DMA,
        ],
        compiler_params=pltpu.CompilerParams(needs_layout_passes=False),
    )
    def k(task_id_hbm, db_id_hbm, task_t_hbm, db_t_hbm,
          out_task_hbm, out_db_hbm,
          row_v, dbrow_v, idx_v, out0_v, out1_v,
          srow, sidx, sout0, sout1):
        wid = lax.axis_index("s") * NC + lax.axis_index("c")
        outs = (out0_v, out1_v)
        souts = (sout0, sout1)
        pending = [None, None]

        def gather_quarter(buf, h, ob):
            out_v = outs[ob]

            @plsc.parallel_loop(0, QTR // LANES, unroll=UNROLL)
            def body(q):
                iv = idx_v[pl.ds(h * QTR + q * LANES, LANES)]
                out_v[pl.ds(q * LANES, LANES)] = plsc.load_gather(buf, [iv])

        def lookup_dim(buf, out_hbm, d):
            for h in range(4):
                ob = h & 1
                if pending[ob] is not None:
                    pending[ob].wait()
                gather_quarter(buf, h, ob)
                pending[ob] = pltpu.async_copy(
                    outs[ob], out_hbm.at[d, pl.ds(h * QTR, QTR)], souts[ob])

        c_idx = pltpu.async_copy(task_id_hbm, idx_v, sidx)
        c_row = pltpu.async_copy(task_t_hbm.at[wid], row_v, srow)
        c_idx.wait()
        c_row.wait()
        lookup_dim(row_v, out_task_hbm, wid)

        c_row = pltpu.async_copy(task_t_hbm.at[wid + NW], row_v, srow)
        c_row.wait()
        lookup_dim(row_v, out_task_hbm, wid + NW)

        c_row = pltpu.async_copy(db_t_hbm.at[wid], dbrow_v, srow)
        c_idx = pltpu.async_copy(db_id_hbm, idx_v, sidx)
        c_row.wait()
        c_idx.wait()
        lookup_dim(dbrow_v, out_db_hbm, wid)

        for ob in range(2):
            if pending[ob] is not None:
                pending[ob].wait()

    return k(task_id, db_id, task_t, db_t)


def kernel(task_id, db_id, task_table, db_table):
    out_t_task, out_t_db = _sc_gather_t(
        task_id.astype(jnp.int32), db_id.astype(jnp.int32),
        task_table.T, db_table.T)
    return (out_t_task.T, out_t_db.T)


# speedup vs baseline: 1.0840x; 1.0840x over previous
import functools

import jax
import jax.numpy as jnp
from jax import lax
from jax.experimental import pallas as pl
from jax.experimental.pallas import tpu as pltpu
from jax.experimental.pallas import tpu_sc as plsc

NUM_TASKS = 100000
NUM_DBS = 1000
TASK_DIM = 64
DB_DIM = 32
BATCH = 16384

_INFO = plsc.get_sparse_core_info()
NC = _INFO.num_cores
NS = _INFO.num_subcores
NW = NC * NS
QTR = BATCH // 4
LANES = 16
UNROLL = 8


def _sc_gather_t(task_id, db_id, task_t, db_t):
    mesh = plsc.VectorSubcoreMesh(core_axis_name="c", subcore_axis_name="s")

    @functools.partial(
        pl.kernel,
        mesh=mesh,
        out_type=(
            jax.ShapeDtypeStruct((TASK_DIM, BATCH), jnp.float32),
            jax.ShapeDtypeStruct((DB_DIM, BATCH), jnp.float32),
        ),
        scratch_types=[
            pltpu.VMEM((NUM_TASKS,), jnp.float32),
            pltpu.VMEM((NUM_DBS,), jnp.float32),
            pltpu.VMEM((BATCH,), jnp.int32),
            pltpu.VMEM((QTR,), jnp.float32),
            pltpu.VMEM((QTR,), jnp.float32),
            pltpu.SemaphoreType.DMA,
            pltpu.SemaphoreType.DMA,
            pltpu.SemaphoreType.DMA,
            pltpu.SemaphoreType.DMA,
            pltpu.SemaphoreType.DMA,
        ],
        compiler_params=pltpu.CompilerParams(
            needs_layout_passes=False,
            disable_bounds_checks=True,
            disable_semaphore_checks=True,
        ),
    )
    def k(task_id_hbm, db_id_hbm, task_t_hbm, db_t_hbm,
          out_task_hbm, out_db_hbm,
          row_v, dbrow_v, idx_v, out0_v, out1_v,
          srow, sidx, sdb, sout0, sout1):
        wid = lax.axis_index("s") * NC + lax.axis_index("c")
        outs = (out0_v, out1_v)
        souts = (sout0, sout1)
        pending = [None, None]

        def gather_quarter(buf, h, ob):
            out_v = outs[ob]

            @plsc.parallel_loop(0, QTR // LANES, unroll=UNROLL)
            def body(q):
                iv = idx_v[pl.ds(h * QTR + q * LANES, LANES)]
                out_v[pl.ds(q * LANES, LANES)] = plsc.load_gather(buf, [iv])

        def lookup_dim(buf, out_hbm, d):
            for h in range(4):
                ob = h & 1
                if pending[ob] is not None:
                    pending[ob].wait()
                gather_quarter(buf, h, ob)
                pending[ob] = pltpu.async_copy(
                    outs[ob], out_hbm.at[d, pl.ds(h * QTR, QTR)], souts[ob])

        c_row = pltpu.async_copy(task_t_hbm.at[wid], row_v, srow)
        c_dbrow = pltpu.async_copy(db_t_hbm.at[wid], dbrow_v, sdb)
        c_idx = pltpu.async_copy(db_id_hbm, idx_v, sidx)
        c_dbrow.wait()
        c_idx.wait()
        lookup_dim(dbrow_v, out_db_hbm, wid)

        c_idx = pltpu.async_copy(task_id_hbm, idx_v, sidx)
        c_idx.wait()
        c_row.wait()
        lookup_dim(row_v, out_task_hbm, wid)

        c_row = pltpu.async_copy(task_t_hbm.at[wid + NW], row_v, srow)
        c_row.wait()
        lookup_dim(row_v, out_task_hbm, wid + NW)

        for ob in range(2):
            if pending[ob] is not None:
                pending[ob].wait()

    return k(task_id, db_id, task_t, db_t)


def kernel(task_id, db_id, task_table, db_table):
    out_t_task, out_t_db = _sc_gather_t(
        task_id.astype(jnp.int32), db_id.astype(jnp.int32),
        task_table.T, db_table.T)
    return (out_t_task.T, out_t_db.T)
